# Initial kernel scaffold; baseline (speedup 1.0000x reference)
import jax
import jax.numpy as jnp
from jax.experimental import pallas as pl


def kernel(x, where):
    raise NotImplementedError("write your pallas kernel here")


# speedup vs baseline: 31.4009x; 31.4009x over previous
import functools

import jax
import jax.numpy as jnp
from jax import lax
from jax.experimental import pallas as pl
from jax.experimental.pallas import tpu as pltpu
from jax.experimental.pallas import tpu_sc as plsc


def kernel(x, where):
    N, C, H, W = x.shape
    OH, OW = 2 * H, 2 * W

    info = plsc.get_sparse_core_info()
    num_cores, num_subcores, L = info.num_cores, info.num_subcores, info.num_lanes
    NW = num_cores * num_subcores

    rows = N * C * H
    rows_per_w = rows // NW
    CHUNK = 128
    n_chunks = rows_per_w // CHUNK
    assert rows % NW == 0 and rows_per_w % CHUNK == 0
    assert W % L == 0 or (2 * W) % L == 0

    in_chunk = CHUNK * W
    out_chunk = CHUNK * 4 * W

    xf = x.reshape(-1)
    wf = where.reshape(-1)
    mesh = plsc.VectorSubcoreMesh(core_axis_name="c", subcore_axis_name="s")

    @functools.partial(
        pl.kernel,
        out_type=jax.ShapeDtypeStruct((rows * 4 * W,), jnp.float32),
        mesh=mesh,
        scratch_types=[
            pltpu.VMEM((in_chunk,), jnp.float32),
            pltpu.VMEM((in_chunk,), jnp.int32),
            pltpu.VMEM((out_chunk,), jnp.float32),
        ],
        compiler_params=pltpu.CompilerParams(
            use_tc_tiling_on_sc=False, needs_layout_passes=False
        ),
    )
    def unpool(x_hbm, w_hbm, out_hbm, x_v, w_v, out_v):
        wid = lax.axis_index("s") * num_cores + lax.axis_index("c")
        iota = lax.iota(jnp.int32, L)
        iota2 = iota * 2
        zero = jnp.zeros((L,), jnp.float32)
        base_in = wid * rows_per_w * W

        def chunk_body(ci, carry):
            off_in = base_in + ci * in_chunk
            pltpu.sync_copy(x_hbm.at[pl.ds(off_in, in_chunk)], x_v)
            pltpu.sync_copy(w_hbm.at[pl.ds(off_in, in_chunk)], w_v)

            def zero_body(k, c2):
                out_v[pl.ds(k * L, L)] = zero
                return c2

            lax.fori_loop(0, out_chunk // L, zero_body, 0)

            def row_body(i, c2):
                xb = i * W
                ob = i * (4 * W)
                for u in range(W // L):
                    xv = x_v[pl.ds(xb + u * L, L)]
                    wv = w_v[pl.ds(xb + u * L, L)]
                    idxv = (wv >> 1) * (2 * W) + (wv & 1) + (iota2 + (ob + 2 * u * L))
                    plsc.store_scatter(out_v, [idxv], xv)
                return c2

            lax.fori_loop(0, CHUNK, row_body, 0)
            pltpu.sync_copy(out_v, out_hbm.at[pl.ds(off_in * 4, out_chunk)])
            return carry

        lax.fori_loop(0, n_chunks, chunk_body, 0)

    out = unpool(xf, wf)
    return out.reshape(N, C, OH, OW)


# speedup vs baseline: 60.8861x; 1.9390x over previous
import functools

import jax
import jax.numpy as jnp
from jax import lax
from jax.experimental import pallas as pl
from jax.experimental.pallas import tpu as pltpu
from jax.experimental.pallas import tpu_sc as plsc


def kernel(x, where):
    N, C, H, W = x.shape
    OH, OW = 2 * H, 2 * W

    info = plsc.get_sparse_core_info()
    num_cores, num_subcores, L = info.num_cores, info.num_subcores, info.num_lanes
    NW = num_cores * num_subcores

    rows = N * C * H
    rows_per_w = rows // NW
    CHUNK = 128
    n_chunks = rows_per_w // CHUNK
    assert rows % NW == 0 and rows_per_w % CHUNK == 0
    assert (2 * W) % L == 0

    in_chunk = CHUNK * W
    out_chunk = CHUNK * 4 * W
    UPV = 2 * W // L

    xf = x.reshape(-1)
    wf = where.reshape(-1)
    mesh = plsc.VectorSubcoreMesh(core_axis_name="c", subcore_axis_name="s")

    @functools.partial(
        pl.kernel,
        out_type=jax.ShapeDtypeStruct((rows * 4 * W,), jnp.float32),
        mesh=mesh,
        scratch_types=[
            pltpu.VMEM((in_chunk,), jnp.float32),
            pltpu.VMEM((in_chunk,), jnp.float32),
            pltpu.VMEM((in_chunk,), jnp.int32),
            pltpu.VMEM((in_chunk,), jnp.int32),
            pltpu.VMEM((out_chunk,), jnp.float32),
            pltpu.VMEM((out_chunk,), jnp.float32),
            pltpu.SemaphoreType.DMA,
            pltpu.SemaphoreType.DMA,
            pltpu.SemaphoreType.DMA,
            pltpu.SemaphoreType.DMA,
            pltpu.SemaphoreType.DMA,
            pltpu.SemaphoreType.DMA,
        ],
        compiler_params=pltpu.CompilerParams(
            use_tc_tiling_on_sc=False, needs_layout_passes=False
        ),
    )
    def unpool(x_hbm, w_hbm, out_hbm,
               x_v0, x_v1, w_v0, w_v1, o_v0, o_v1,
               sx0, sx1, sw0, sw1, so0, so1):
        wid = lax.axis_index("s") * num_cores + lax.axis_index("c")
        iota = lax.iota(jnp.int32, L)
        J = [(iota >> 1) + u * (L // 2) for u in range(UPV)]
        t0 = iota & 1
        t1 = t0 + 2
        zero = jnp.zeros((L,), jnp.float32)
        base_in = wid * rows_per_w * W

        bufs = [(x_v0, w_v0, o_v0, sx0, sw0, so0),
                (x_v1, w_v1, o_v1, sx1, sw1, so1)]

        def start_in(ci, b):
            off = base_in + ci * in_chunk
            x_vb, w_vb = bufs[b][0], bufs[b][1]
            cx = pltpu.async_copy(x_hbm.at[pl.ds(off, in_chunk)], x_vb, bufs[b][3])
            cw = pltpu.async_copy(w_hbm.at[pl.ds(off, in_chunk)], w_vb, bufs[b][4])
            return cx, cw

        def start_out(ci, b):
            off = (base_in + ci * in_chunk) * 4
            return pltpu.async_copy(
                bufs[b][2], out_hbm.at[pl.ds(off, out_chunk)], bufs[b][5]
            )

        def compute(b):
            x_vb, w_vb, o_vb = bufs[b][0], bufs[b][1], bufs[b][2]

            @plsc.parallel_loop(0, CHUNK, step=1, unroll=2)
            def _row(i):
                x_row = x_vb.at[pl.ds(i * W, W)]
                w_row = w_vb.at[pl.ds(i * W, W)]
                ob = i * (4 * W)
                for u in range(UPV):
                    xv = plsc.load_gather(x_row, [J[u]])
                    wv = plsc.load_gather(w_row, [J[u]])
                    o_vb[pl.ds(ob + u * L, L)] = jnp.where(wv == t0, xv, zero)
                    o_vb[pl.ds(ob + 2 * W + u * L, L)] = jnp.where(wv == t1, xv, zero)

        in_d = {0: start_in(0, 0)}
        if n_chunks > 1:
            in_d[1] = start_in(1, 1)
        out_d = {}
        for ci in range(n_chunks):
            b = ci % 2
            cx, cw = in_d.pop(ci)
            cx.wait()
            cw.wait()
            if ci - 2 >= 0:
                out_d.pop(ci - 2).wait()
            compute(b)
            out_d[ci] = start_out(ci, b)
            if ci + 2 < n_chunks:
                in_d[ci + 2] = start_in(ci + 2, b)
        for d in out_d.values():
            d.wait()

    out = unpool(xf, wf)
    return out.reshape(N, C, OH, OW)


# speedup vs baseline: 81.3823x; 1.3366x over previous
import functools

import jax
import jax.numpy as jnp
from jax import lax
from jax.experimental import pallas as pl
from jax.experimental.pallas import tpu as pltpu
from jax.experimental.pallas import tpu_sc as plsc


def kernel(x, where):
    N, C, H, W = x.shape
    OH, OW = 2 * H, 2 * W

    info = plsc.get_sparse_core_info()
    num_cores, num_subcores, L = info.num_cores, info.num_subcores, info.num_lanes
    NW = num_cores * num_subcores

    planes = N * C
    planes_per_w = planes // NW
    assert planes % NW == 0
    assert (2 * W) % L == 0

    in_plane = H * W
    UPV = 2 * W // L

    xf = x.reshape(-1)
    wf = where.reshape(-1)
    mesh = plsc.VectorSubcoreMesh(core_axis_name="c", subcore_axis_name="s")

    @functools.partial(
        pl.kernel,
        out_type=jax.ShapeDtypeStruct((N, C, OH, OW), jnp.float32),
        mesh=mesh,
        scratch_types=[
            pltpu.VMEM((in_plane,), jnp.float32),
            pltpu.VMEM((in_plane,), jnp.float32),
            pltpu.VMEM((in_plane,), jnp.int32),
            pltpu.VMEM((in_plane,), jnp.int32),
            pltpu.VMEM((OH, OW), jnp.float32),
            pltpu.VMEM((OH, OW), jnp.float32),
            pltpu.SemaphoreType.DMA,
            pltpu.SemaphoreType.DMA,
            pltpu.SemaphoreType.DMA,
            pltpu.SemaphoreType.DMA,
            pltpu.SemaphoreType.DMA,
            pltpu.SemaphoreType.DMA,
        ],
        compiler_params=pltpu.CompilerParams(
            use_tc_tiling_on_sc=True, needs_layout_passes=False
        ),
    )
    def unpool(x_hbm, w_hbm, out_hbm,
               x_v0, x_v1, w_v0, w_v1, o_v0, o_v1,
               sx0, sx1, sw0, sw1, so0, so1):
        wid = lax.axis_index("s") * num_cores + lax.axis_index("c")
        iota = lax.iota(jnp.int32, L)
        J = [(iota >> 1) + u * (L // 2) for u in range(UPV)]
        t0 = iota & 1
        t1 = t0 + 2
        zero = jnp.zeros((L,), jnp.float32)
        plane0 = wid * planes_per_w

        bufs = [(x_v0, w_v0, o_v0, sx0, sw0, so0),
                (x_v1, w_v1, o_v1, sx1, sw1, so1)]

        def start_in(k, b):
            off = (plane0 + k) * in_plane
            cx = pltpu.async_copy(x_hbm.at[pl.ds(off, in_plane)], bufs[b][0], bufs[b][3])
            cw = pltpu.async_copy(w_hbm.at[pl.ds(off, in_plane)], bufs[b][1], bufs[b][4])
            return cx, cw

        def start_out(k, b):
            p = plane0 + k
            n = p // C
            c = p % C
            return pltpu.async_copy(bufs[b][2], out_hbm.at[n, c], bufs[b][5])

        def compute(b):
            x_vb, w_vb, o_vb = bufs[b][0], bufs[b][1], bufs[b][2]

            @plsc.parallel_loop(0, H, step=1, unroll=2)
            def _row(i):
                x_row = x_vb.at[pl.ds(i * W, W)]
                w_row = w_vb.at[pl.ds(i * W, W)]
                for u in range(UPV):
                    xv = plsc.load_gather(x_row, [J[u]])
                    wv = plsc.load_gather(w_row, [J[u]])
                    o_vb[2 * i, pl.ds(u * L, L)] = jnp.where(wv == t0, xv, zero)
                    o_vb[2 * i + 1, pl.ds(u * L, L)] = jnp.where(wv == t1, xv, zero)

        in_d = {0: start_in(0, 0)}
        if planes_per_w > 1:
            in_d[1] = start_in(1, 1)
        out_d = {}

        def plane_step(k, b):
            cx, cw = in_d.pop(k)
            cx.wait()
            cw.wait()
            if k - 2 >= 0:
                out_d.pop(k - 2).wait()
            compute(b)
            out_d[k] = start_out(k, b)
            if k + 2 < planes_per_w:
                in_d[k + 2] = start_in(k + 2, b)

        for k in range(planes_per_w):
            plane_step(k, k % 2)
        for d in out_d.values():
            d.wait()

    out = unpool(xf, wf)
    return out


# speedup vs baseline: 90.9422x; 1.1175x over previous
import functools

import jax
import jax.numpy as jnp
from jax import lax
from jax.experimental import pallas as pl
from jax.experimental.pallas import tpu as pltpu
from jax.experimental.pallas import tpu_sc as plsc


def kernel(x, where):
    N, C, H, W = x.shape
    OH, OW = 2 * H, 2 * W

    info = plsc.get_sparse_core_info()
    num_cores, num_subcores, L = info.num_cores, info.num_subcores, info.num_lanes
    NW = num_cores * num_subcores

    planes = N * C
    planes_per_w = planes // NW
    assert planes % NW == 0
    assert (2 * W) % L == 0

    in_plane = H * W
    UPV = 2 * W // L

    xf = x
    wf = where
    mesh = plsc.VectorSubcoreMesh(core_axis_name="c", subcore_axis_name="s")

    @functools.partial(
        pl.kernel,
        out_type=jax.ShapeDtypeStruct((N, C, OH, OW), jnp.float32),
        mesh=mesh,
        scratch_types=[
            pltpu.VMEM((H, W), jnp.float32),
            pltpu.VMEM((H, W), jnp.float32),
            pltpu.VMEM((H, W), jnp.int32),
            pltpu.VMEM((H, W), jnp.int32),
            pltpu.VMEM((OH, OW), jnp.float32),
            pltpu.VMEM((OH, OW), jnp.float32),
            pltpu.SemaphoreType.DMA,
            pltpu.SemaphoreType.DMA,
            pltpu.SemaphoreType.DMA,
            pltpu.SemaphoreType.DMA,
            pltpu.SemaphoreType.DMA,
            pltpu.SemaphoreType.DMA,
        ],
        compiler_params=pltpu.CompilerParams(
            use_tc_tiling_on_sc=True, needs_layout_passes=False
        ),
    )
    def unpool(x_hbm, w_hbm, out_hbm,
               x_v0, x_v1, w_v0, w_v1, o_v0, o_v1,
               sx0, sx1, sw0, sw1, so0, so1):
        wid = lax.axis_index("s") * num_cores + lax.axis_index("c")
        iota = lax.iota(jnp.int32, L)
        J = [(iota >> 1) + u * (L // 2) for u in range(UPV)]
        t0 = iota & 1
        t1 = t0 + 2
        zero = jnp.zeros((L,), jnp.float32)
        plane0 = wid * planes_per_w

        bufs = [(x_v0, w_v0, o_v0, sx0, sw0, so0),
                (x_v1, w_v1, o_v1, sx1, sw1, so1)]

        def start_in(k, b):
            p = plane0 + k
            n = p // C
            c = p % C
            cx = pltpu.async_copy(x_hbm.at[n, c], bufs[b][0], bufs[b][3])
            cw = pltpu.async_copy(w_hbm.at[n, c], bufs[b][1], bufs[b][4])
            return cx, cw

        def start_out(k, b):
            p = plane0 + k
            n = p // C
            c = p % C
            return pltpu.async_copy(bufs[b][2], out_hbm.at[n, c], bufs[b][5])

        def compute(b):
            x_vb, w_vb, o_vb = bufs[b][0], bufs[b][1], bufs[b][2]

            @plsc.parallel_loop(0, H, step=1, unroll=2)
            def _row(i):
                x_row = x_vb.at[i]
                w_row = w_vb.at[i]
                for u in range(UPV):
                    xv = plsc.load_gather(x_row, [J[u]])
                    wv = plsc.load_gather(w_row, [J[u]])
                    o_vb[2 * i, pl.ds(u * L, L)] = jnp.where(wv == t0, xv, zero)
                    o_vb[2 * i + 1, pl.ds(u * L, L)] = jnp.where(wv == t1, xv, zero)

        in_d = {0: start_in(0, 0)}
        if planes_per_w > 1:
            in_d[1] = start_in(1, 1)
        out_d = {}

        def plane_step(k, b):
            cx, cw = in_d.pop(k)
            cx.wait()
            cw.wait()
            if k - 2 >= 0:
                out_d.pop(k - 2).wait()
            compute(b)
            out_d[k] = start_out(k, b)
            if k + 2 < planes_per_w:
                in_d[k + 2] = start_in(k + 2, b)

        for k in range(planes_per_w):
            plane_step(k, k % 2)
        for d in out_d.values():
            d.wait()

    out = unpool(xf, wf)
    return out


# speedup vs baseline: 92.3927x; 1.0159x over previous
import functools

import jax
import jax.numpy as jnp
from jax import lax
from jax.experimental import pallas as pl
from jax.experimental.pallas import tpu as pltpu
from jax.experimental.pallas import tpu_sc as plsc


def kernel(x, where):
    N, C, H, W = x.shape
    OH, OW = 2 * H, 2 * W

    info = plsc.get_sparse_core_info()
    num_cores, num_subcores, L = info.num_cores, info.num_subcores, info.num_lanes
    NW = num_cores * num_subcores

    planes = N * C
    planes_per_w = planes // NW
    PCHUNK = 2
    assert planes % NW == 0
    assert planes_per_w % PCHUNK == 0 and C % planes_per_w == 0
    assert (2 * W) % L == 0
    n_steps = planes_per_w // PCHUNK

    in_plane = H * W
    UPV = 2 * W // L

    xf = x
    wf = where
    mesh = plsc.VectorSubcoreMesh(core_axis_name="c", subcore_axis_name="s")

    @functools.partial(
        pl.kernel,
        out_type=jax.ShapeDtypeStruct((N, C, OH, OW), jnp.float32),
        mesh=mesh,
        scratch_types=[
            pltpu.VMEM((PCHUNK, H, W), jnp.float32),
            pltpu.VMEM((PCHUNK, H, W), jnp.float32),
            pltpu.VMEM((PCHUNK, H, W), jnp.int32),
            pltpu.VMEM((PCHUNK, H, W), jnp.int32),
            pltpu.VMEM((PCHUNK, OH, OW), jnp.float32),
            pltpu.VMEM((PCHUNK, OH, OW), jnp.float32),
            pltpu.SemaphoreType.DMA,
            pltpu.SemaphoreType.DMA,
            pltpu.SemaphoreType.DMA,
            pltpu.SemaphoreType.DMA,
            pltpu.SemaphoreType.DMA,
            pltpu.SemaphoreType.DMA,
        ],
        compiler_params=pltpu.CompilerParams(
            use_tc_tiling_on_sc=True, needs_layout_passes=False
        ),
    )
    def unpool(x_hbm, w_hbm, out_hbm,
               x_v0, x_v1, w_v0, w_v1, o_v0, o_v1,
               sx0, sx1, sw0, sw1, so0, so1):
        wid = lax.axis_index("s") * num_cores + lax.axis_index("c")
        iota = lax.iota(jnp.int32, L)
        J = [(iota >> 1) + u * (L // 2) for u in range(UPV)]
        t0 = iota & 1
        t1 = t0 + 2
        zero = jnp.zeros((L,), jnp.float32)
        plane0 = wid * planes_per_w

        bufs = [(x_v0, w_v0, o_v0, sx0, sw0, so0),
                (x_v1, w_v1, o_v1, sx1, sw1, so1)]

        def start_in(k, b):
            p = plane0 + k * PCHUNK
            n = p // C
            c = p % C
            cx = pltpu.async_copy(x_hbm.at[n, pl.ds(c, PCHUNK)], bufs[b][0], bufs[b][3])
            cw = pltpu.async_copy(w_hbm.at[n, pl.ds(c, PCHUNK)], bufs[b][1], bufs[b][4])
            return cx, cw

        def start_out(k, b):
            p = plane0 + k * PCHUNK
            n = p // C
            c = p % C
            return pltpu.async_copy(bufs[b][2], out_hbm.at[n, pl.ds(c, PCHUNK)], bufs[b][5])

        def compute(b):
            x_vb, w_vb, o_vb = bufs[b][0], bufs[b][1], bufs[b][2]

            for q in range(PCHUNK):
                @plsc.parallel_loop(0, H, step=1, unroll=2)
                def _row(i, q=q):
                    x_row = x_vb.at[q, i]
                    w_row = w_vb.at[q, i]
                    for u in range(UPV):
                        xv = plsc.load_gather(x_row, [J[u]])
                        wv = plsc.load_gather(w_row, [J[u]])
                        o_vb[q, 2 * i, pl.ds(u * L, L)] = jnp.where(wv == t0, xv, zero)
                        o_vb[q, 2 * i + 1, pl.ds(u * L, L)] = jnp.where(wv == t1, xv, zero)

        in_d = {0: start_in(0, 0)}
        if n_steps > 1:
            in_d[1] = start_in(1, 1)
        out_d = {}

        def plane_step(k, b):
            cx, cw = in_d.pop(k)
            cx.wait()
            cw.wait()
            if k - 2 >= 0:
                out_d.pop(k - 2).wait()
            compute(b)
            out_d[k] = start_out(k, b)
            if k + 2 < n_steps:
                in_d[k + 2] = start_in(k + 2, b)

        for k in range(n_steps):
            plane_step(k, k % 2)
        for d in out_d.values():
            d.wait()

    out = unpool(xf, wf)
    return out
